# baseline (speedup 1.0000x reference)
import functools

import jax
import jax.numpy as jnp
from jax import lax
from jax.experimental import pallas as pl
from jax.experimental.pallas import tpu as pltpu
from jax.experimental.pallas import tpu_sc as plsc

N = 10000
E = 320000
D = 128
EPS = 1e-5

NCORES = 2
NSUB = 16
NW = NCORES * NSUB
NP = 10016
ROWS_PER_SUB = NP // NSUB
IDXN = 128
IPW = 80
DEGN = 1024
DPW = 10
E_PAD = NW * IDXN * IPW
DEG_W = 16

ROW_BLK = 2000


def _proj_rows(x):
    d2 = jnp.sum(x * x, axis=1, keepdims=True)
    norm = jnp.sqrt(d2 + 1e-15)
    max_norm = 1.0 - EPS
    scale = jnp.where(norm > max_norm, max_norm / norm, jnp.ones_like(norm))
    return x * scale


def _pre_body(x_ref, w_ref, b_ref, h_ref, *, project):
    x = x_ref[...]
    if project:
        x = _proj_rows(x)
    h_ref[...] = (
        jnp.dot(x, w_ref[...], preferred_element_type=jnp.float32,
                precision=lax.Precision.HIGHEST)
        + b_ref[...]
    )


def _tc_pre(x, w, b, project):
    return pl.pallas_call(
        functools.partial(_pre_body, project=project),
        grid=(N // ROW_BLK,),
        in_specs=[
            pl.BlockSpec((ROW_BLK, D), lambda i: (i, 0)),
            pl.BlockSpec((D, D), lambda i: (0, 0)),
            pl.BlockSpec((1, D), lambda i: (0, 0)),
        ],
        out_specs=pl.BlockSpec((ROW_BLK, D), lambda i: (i, 0)),
        out_shape=jax.ShapeDtypeStruct((N, D), jnp.float32),
    )(x, w, b.reshape(1, D))


def _post_body(h_ref, p_ref, d_ref, o_ref, *, activation):
    h = h_ref[...]
    agg = p_ref[0] + p_ref[1]
    deg = d_ref[0][:, 0:1] + d_ref[1][:, 0:1]
    deg = jnp.maximum(deg, 1.0)
    out = _proj_rows(0.5 * (h + agg / deg))
    if activation:
        d2 = jnp.sum(out * out, axis=1, keepdims=True)
        denom = jnp.clip(1.0 - d2, 1e-7, None)
        arg = jnp.clip(1.0 + 2.0 * d2 / denom, 1.0 + 1e-7, None)
        nrm = jnp.log(arg + jnp.sqrt(arg * arg - 1.0))
        sig = 1.0 / (1.0 + jnp.exp(-nrm))
        out = _proj_rows(out * sig)
    o_ref[...] = out


def _tc_post(h, agg_p, deg_p, activation):
    return pl.pallas_call(
        functools.partial(_post_body, activation=activation),
        grid=(N // ROW_BLK,),
        in_specs=[
            pl.BlockSpec((ROW_BLK, D), lambda i: (i, 0)),
            pl.BlockSpec((NCORES, ROW_BLK, D), lambda i: (0, i, 0)),
            pl.BlockSpec((NCORES, ROW_BLK, DEG_W), lambda i: (0, i, 0)),
        ],
        out_specs=pl.BlockSpec((ROW_BLK, D), lambda i: (i, 0)),
        out_shape=jax.ShapeDtypeStruct((N, D), jnp.float32),
    )(h, agg_p, deg_p)


def _sc_agg_body(h_hbm, ed_hbm, zagg_hbm, agg_out, idx_v, rows_v,
                 acc_sh, gsem):
    c = lax.axis_index("c")
    s = lax.axis_index("s")
    wid = c * NSUB + s
    r0 = s * ROWS_PER_SUB
    pltpu.sync_copy(zagg_hbm, acc_sh.at[pl.ds(r0, ROWS_PER_SUB)])
    plsc.subcore_barrier()

    t0 = wid * IPW

    def body(j, carry):
        pltpu.sync_copy(ed_hbm.at[t0 + j], idx_v)
        pltpu.async_copy(h_hbm.at[idx_v.at[0]], rows_v, gsem).wait()
        pltpu.sync_copy(rows_v, acc_sh.at[idx_v.at[1]], add=True)
        return carry

    lax.fori_loop(0, IPW, body, 0)
    plsc.subcore_barrier()
    out_base = c * NP + r0
    pltpu.sync_copy(acc_sh.at[pl.ds(r0, ROWS_PER_SUB)],
                    agg_out.at[pl.ds(out_base, ROWS_PER_SUB)])


def _sc_deg_body(dsti_hbm, zdeg_hbm, ones_hbm, deg_out, idx_v, ones_v,
                 dacc_sh):
    c = lax.axis_index("c")
    s = lax.axis_index("s")
    wid = c * NSUB + s
    r0 = s * ROWS_PER_SUB
    pltpu.sync_copy(zdeg_hbm, dacc_sh.at[pl.ds(r0, ROWS_PER_SUB)])
    pltpu.sync_copy(ones_hbm, ones_v)
    plsc.subcore_barrier()

    t0 = wid * DPW

    def body(j, carry):
        pltpu.sync_copy(dsti_hbm.at[t0 + j], idx_v.at[0])
        pltpu.sync_copy(ones_v, dacc_sh.at[idx_v.at[0]], add=True)
        return carry

    lax.fori_loop(0, DPW, body, 0)
    plsc.subcore_barrier()
    out_base = c * NP + r0
    pltpu.sync_copy(dacc_sh.at[pl.ds(r0, ROWS_PER_SUB)],
                    deg_out.at[pl.ds(out_base, ROWS_PER_SUB)])


@functools.cache
def _sc_agg():
    mesh = plsc.VectorSubcoreMesh(core_axis_name="c", subcore_axis_name="s",
                                  num_cores=NCORES, num_subcores=NSUB)
    return pl.kernel(
        _sc_agg_body,
        out_type=jax.ShapeDtypeStruct((NCORES * NP, D), jnp.float32),
        mesh=mesh,
        compiler_params=pltpu.CompilerParams(use_tc_tiling_on_sc=False),
        scratch_types=[
            pltpu.VMEM((2, IDXN), jnp.int32),
            pltpu.VMEM((IDXN, D), jnp.float32),
            pltpu.VMEM_SHARED((NP, D), jnp.float32),
            pltpu.SemaphoreType.DMA,
        ],
    )


@functools.cache
def _sc_deg():
    mesh = plsc.VectorSubcoreMesh(core_axis_name="c", subcore_axis_name="s",
                                  num_cores=NCORES, num_subcores=NSUB)
    return pl.kernel(
        _sc_deg_body,
        out_type=jax.ShapeDtypeStruct((NCORES * NP, DEG_W), jnp.float32),
        mesh=mesh,
        compiler_params=pltpu.CompilerParams(use_tc_tiling_on_sc=False),
        scratch_types=[
            pltpu.VMEM((1, DEGN), jnp.int32),
            pltpu.VMEM((DEGN, DEG_W), jnp.float32),
            pltpu.VMEM_SHARED((NP, DEG_W), jnp.float32),
        ],
    )


def kernel(x, edge_index, W1, b1, W2, b2):
    src = edge_index[0].astype(jnp.int32)
    dst = edge_index[1].astype(jnp.int32)
    pad = E_PAD - E
    src = jnp.concatenate([src, jnp.zeros((pad,), jnp.int32)])
    dst = jnp.concatenate([dst, jnp.full((pad,), N, jnp.int32)])
    ed = jnp.stack([src.reshape(-1, IDXN), dst.reshape(-1, IDXN)], axis=1)
    dsti = dst.reshape(-1, DEGN)
    zagg = jnp.zeros((ROWS_PER_SUB, D), jnp.float32)
    zdeg = jnp.zeros((ROWS_PER_SUB, DEG_W), jnp.float32)
    ones = jnp.ones((DEGN, DEG_W), jnp.float32)

    deg = _sc_deg()(dsti, zdeg, ones)
    deg_p = deg.reshape(NCORES, NP, DEG_W)
    h1 = _tc_pre(x, W1, b1, project=True)
    agg1 = _sc_agg()(h1, ed, zagg)
    y1 = _tc_post(h1, agg1.reshape(NCORES, NP, D), deg_p, activation=True)
    h2 = _tc_pre(y1, W2, b2, project=False)
    agg2 = _sc_agg()(h2, ed, zagg)
    out = _tc_post(h2, agg2.reshape(NCORES, NP, D), deg_p, activation=False)
    return out


# speedup vs baseline: 1.1439x; 1.1439x over previous
import functools

import jax
import jax.numpy as jnp
from jax import lax
from jax.experimental import pallas as pl
from jax.experimental.pallas import tpu as pltpu
from jax.experimental.pallas import tpu_sc as plsc

N = 10000
E = 320000
D = 128
EPS = 1e-5

NCORES = 2
NSUB = 16
NW = NCORES * NSUB
NP = 10016
ROWS_PER_SUB = NP // NSUB
IDXN = 128
IPW_HEAVY = 114
IPW_LIGHT = 46
TOT_CH = NSUB * (IPW_HEAVY + IPW_LIGHT)
DEGN = 1024
DPW = 10
E_PAD = TOT_CH * IDXN
DEG_W = 16

ROW_BLK = 2000


def _proj_rows(x):
    d2 = jnp.sum(x * x, axis=1, keepdims=True)
    norm = jnp.sqrt(d2 + 1e-15)
    max_norm = 1.0 - EPS
    scale = jnp.where(norm > max_norm, max_norm / norm, jnp.ones_like(norm))
    return x * scale


def _pre_body(x_ref, w_ref, b_ref, h_ref, *, project):
    x = x_ref[...]
    if project:
        x = _proj_rows(x)
    h_ref[...] = (
        jnp.dot(x, w_ref[...], preferred_element_type=jnp.float32,
                precision=lax.Precision.HIGHEST)
        + b_ref[...]
    )


def _tc_pre(x, w, b, project):
    return pl.pallas_call(
        functools.partial(_pre_body, project=project),
        grid=(N // ROW_BLK,),
        in_specs=[
            pl.BlockSpec((ROW_BLK, D), lambda i: (i, 0)),
            pl.BlockSpec((D, D), lambda i: (0, 0)),
            pl.BlockSpec((1, D), lambda i: (0, 0)),
        ],
        out_specs=pl.BlockSpec((ROW_BLK, D), lambda i: (i, 0)),
        out_shape=jax.ShapeDtypeStruct((N, D), jnp.float32),
    )(x, w, b.reshape(1, D))


def _post_body(h_ref, p_ref, d_ref, o_ref, *, activation):
    h = h_ref[...]
    agg = p_ref[0] + p_ref[1]
    deg = d_ref[0][:, 0:1] + d_ref[1][:, 0:1]
    deg = jnp.maximum(deg, 1.0)
    out = _proj_rows(0.5 * (h + agg / deg))
    if activation:
        d2 = jnp.sum(out * out, axis=1, keepdims=True)
        denom = jnp.clip(1.0 - d2, 1e-7, None)
        arg = jnp.clip(1.0 + 2.0 * d2 / denom, 1.0 + 1e-7, None)
        nrm = jnp.log(arg + jnp.sqrt(arg * arg - 1.0))
        sig = 1.0 / (1.0 + jnp.exp(-nrm))
        out = _proj_rows(out * sig)
    o_ref[...] = out


def _tc_post(h, agg_p, deg_p, activation):
    return pl.pallas_call(
        functools.partial(_post_body, activation=activation),
        grid=(N // ROW_BLK,),
        in_specs=[
            pl.BlockSpec((ROW_BLK, D), lambda i: (i, 0)),
            pl.BlockSpec((NCORES, ROW_BLK, D), lambda i: (0, i, 0)),
            pl.BlockSpec((NCORES, ROW_BLK, DEG_W), lambda i: (0, i, 0)),
        ],
        out_specs=pl.BlockSpec((ROW_BLK, D), lambda i: (i, 0)),
        out_shape=jax.ShapeDtypeStruct((N, D), jnp.float32),
    )(h, agg_p, deg_p)


def _sc_agg_body(h_hbm, ed_hbm, zagg_hbm, agg_out, idx_v, rows_v,
                 acc_sh, gsem):
    c = lax.axis_index("c")
    s = lax.axis_index("s")
    wid = c * NSUB + s
    r0 = s * ROWS_PER_SUB
    pltpu.sync_copy(zagg_hbm, acc_sh.at[pl.ds(r0, ROWS_PER_SUB)])
    plsc.subcore_barrier()

    t0 = jnp.where(c == 0, s * IPW_HEAVY,
                   NSUB * IPW_HEAVY + s * IPW_LIGHT)
    count = jnp.where(c == 0, IPW_HEAVY, IPW_LIGHT)

    def body(j, carry):
        pltpu.sync_copy(ed_hbm.at[t0 + j], idx_v)
        pltpu.async_copy(h_hbm.at[idx_v.at[0]], rows_v, gsem).wait()
        pltpu.sync_copy(rows_v, acc_sh.at[idx_v.at[1]], add=True)
        return carry

    lax.fori_loop(0, count, body, 0)
    plsc.subcore_barrier()
    out_base = c * NP + r0
    pltpu.sync_copy(acc_sh.at[pl.ds(r0, ROWS_PER_SUB)],
                    agg_out.at[pl.ds(out_base, ROWS_PER_SUB)])


def _sc_deg_body(dsti_hbm, zdeg_hbm, ones_hbm, deg_out, idx_v, ones_v,
                 dacc_sh):
    c = lax.axis_index("c")
    s = lax.axis_index("s")
    wid = c * NSUB + s
    r0 = s * ROWS_PER_SUB
    pltpu.sync_copy(zdeg_hbm, dacc_sh.at[pl.ds(r0, ROWS_PER_SUB)])
    pltpu.sync_copy(ones_hbm, ones_v)
    plsc.subcore_barrier()

    t0 = wid * DPW

    def body(j, carry):
        pltpu.sync_copy(dsti_hbm.at[t0 + j], idx_v.at[0])
        pltpu.sync_copy(ones_v, dacc_sh.at[idx_v.at[0]], add=True)
        return carry

    lax.fori_loop(0, DPW, body, 0)
    plsc.subcore_barrier()
    out_base = c * NP + r0
    pltpu.sync_copy(dacc_sh.at[pl.ds(r0, ROWS_PER_SUB)],
                    deg_out.at[pl.ds(out_base, ROWS_PER_SUB)])


@functools.cache
def _sc_agg():
    mesh = plsc.VectorSubcoreMesh(core_axis_name="c", subcore_axis_name="s",
                                  num_cores=NCORES, num_subcores=NSUB)
    return pl.kernel(
        _sc_agg_body,
        out_type=jax.ShapeDtypeStruct((NCORES * NP, D), jnp.float32),
        mesh=mesh,
        compiler_params=pltpu.CompilerParams(use_tc_tiling_on_sc=False),
        scratch_types=[
            pltpu.VMEM((2, IDXN), jnp.int32),
            pltpu.VMEM((IDXN, D), jnp.float32),
            pltpu.VMEM_SHARED((NP, D), jnp.float32),
            pltpu.SemaphoreType.DMA,
        ],
    )


@functools.cache
def _sc_deg():
    mesh = plsc.VectorSubcoreMesh(core_axis_name="c", subcore_axis_name="s",
                                  num_cores=NCORES, num_subcores=NSUB)
    return pl.kernel(
        _sc_deg_body,
        out_type=jax.ShapeDtypeStruct((NCORES * NP, DEG_W), jnp.float32),
        mesh=mesh,
        compiler_params=pltpu.CompilerParams(use_tc_tiling_on_sc=False),
        scratch_types=[
            pltpu.VMEM((1, DEGN), jnp.int32),
            pltpu.VMEM((DEGN, DEG_W), jnp.float32),
            pltpu.VMEM_SHARED((NP, DEG_W), jnp.float32),
        ],
    )


def kernel(x, edge_index, W1, b1, W2, b2):
    src = edge_index[0].astype(jnp.int32)
    dst = edge_index[1].astype(jnp.int32)
    pad = E_PAD - E
    src = jnp.concatenate([src, jnp.zeros((pad,), jnp.int32)])
    dst = jnp.concatenate([dst, jnp.full((pad,), N, jnp.int32)])
    ed = jnp.stack([src.reshape(-1, IDXN), dst.reshape(-1, IDXN)], axis=1)
    dsti = dst.reshape(-1, DEGN)
    zagg = jnp.zeros((ROWS_PER_SUB, D), jnp.float32)
    zdeg = jnp.zeros((ROWS_PER_SUB, DEG_W), jnp.float32)
    ones = jnp.ones((DEGN, DEG_W), jnp.float32)

    deg = _sc_deg()(dsti, zdeg, ones)
    deg_p = deg.reshape(NCORES, NP, DEG_W)
    h1 = _tc_pre(x, W1, b1, project=True)
    agg1 = _sc_agg()(h1, ed, zagg)
    y1 = _tc_post(h1, agg1.reshape(NCORES, NP, D), deg_p, activation=True)
    h2 = _tc_pre(y1, W2, b2, project=False)
    agg2 = _sc_agg()(h2, ed, zagg)
    out = _tc_post(h2, agg2.reshape(NCORES, NP, D), deg_p, activation=False)
    return out


# speedup vs baseline: 1.3935x; 1.2182x over previous
import functools

import jax
import jax.numpy as jnp
from jax import lax
from jax.experimental import pallas as pl
from jax.experimental.pallas import tpu as pltpu
from jax.experimental.pallas import tpu_sc as plsc

N = 10000
E = 320000
D = 128
EPS = 1e-5

NCORES = 2
NSUB = 16
NW = NCORES * NSUB
NP = 10240
ROWS_PER_SUB = NP // NSUB
CHUNK = 128
CHUNKS_PER_W = -(-E // (NW * CHUNK))
E_PAD = NW * CHUNK * CHUNKS_PER_W
DEG_W = 16

ROW_BLK = 2000


def _proj_rows(x):
    d2 = jnp.sum(x * x, axis=1, keepdims=True)
    norm = jnp.sqrt(d2 + 1e-15)
    max_norm = 1.0 - EPS
    scale = jnp.where(norm > max_norm, max_norm / norm, jnp.ones_like(norm))
    return x * scale


def _pre_body(x_ref, w_ref, b_ref, h_ref, *, project):
    x = x_ref[...]
    if project:
        x = _proj_rows(x)
    h_ref[...] = (
        jnp.dot(x, w_ref[...], preferred_element_type=jnp.float32,
                precision=lax.Precision.HIGHEST)
        + b_ref[...]
    )


def _tc_pre(x, w, b, project):
    return pl.pallas_call(
        functools.partial(_pre_body, project=project),
        grid=(N // ROW_BLK,),
        in_specs=[
            pl.BlockSpec((ROW_BLK, D), lambda i: (i, 0)),
            pl.BlockSpec((D, D), lambda i: (0, 0)),
            pl.BlockSpec((1, D), lambda i: (0, 0)),
        ],
        out_specs=pl.BlockSpec((ROW_BLK, D), lambda i: (i, 0)),
        out_shape=jax.ShapeDtypeStruct((N, D), jnp.float32),
    )(x, w, b.reshape(1, D))


def _post_body(h_ref, p_ref, d_ref, o_ref, *, activation):
    h = h_ref[...]
    agg = p_ref[0] + p_ref[1]
    deg = d_ref[0][:, 0:1] + d_ref[1][:, 0:1]
    deg = jnp.maximum(deg, 1.0)
    out = _proj_rows(0.5 * (h + agg / deg))
    if activation:
        d2 = jnp.sum(out * out, axis=1, keepdims=True)
        denom = jnp.clip(1.0 - d2, 1e-7, None)
        arg = jnp.clip(1.0 + 2.0 * d2 / denom, 1.0 + 1e-7, None)
        nrm = jnp.log(arg + jnp.sqrt(arg * arg - 1.0))
        sig = 1.0 / (1.0 + jnp.exp(-nrm))
        out = _proj_rows(out * sig)
    o_ref[...] = out


def _tc_post(h, agg_p, deg_p, activation):
    return pl.pallas_call(
        functools.partial(_post_body, activation=activation),
        grid=(N // ROW_BLK,),
        in_specs=[
            pl.BlockSpec((ROW_BLK, D), lambda i: (i, 0)),
            pl.BlockSpec((NCORES, ROW_BLK, D), lambda i: (0, i, 0)),
            pl.BlockSpec((NCORES, ROW_BLK, DEG_W), lambda i: (0, i, 0)),
        ],
        out_specs=pl.BlockSpec((ROW_BLK, D), lambda i: (i, 0)),
        out_shape=jax.ShapeDtypeStruct((N, D), jnp.float32),
    )(h, agg_p, deg_p)


def _sc_agg_body(h_hbm, src_hbm, dst_hbm, zagg_hbm, zdeg_hbm, ones_hbm,
                 agg_out, deg_out, src_v, dst_v, rows_v, ones_v,
                 acc_sh, dacc_sh, sem):
    c = lax.axis_index("c")
    s = lax.axis_index("s")
    wid = c * NSUB + s
    r0 = s * ROWS_PER_SUB
    pltpu.sync_copy(zagg_hbm, acc_sh.at[pl.ds(r0, ROWS_PER_SUB)])
    pltpu.sync_copy(zdeg_hbm, dacc_sh.at[pl.ds(r0, ROWS_PER_SUB)])
    pltpu.sync_copy(ones_hbm, ones_v)
    plsc.subcore_barrier()
    e0 = wid * (CHUNKS_PER_W * CHUNK)

    def body(j, carry):
        base = e0 + j * CHUNK
        pltpu.sync_copy(src_hbm.at[pl.ds(base, CHUNK)], src_v)
        pltpu.sync_copy(dst_hbm.at[pl.ds(base, CHUNK)], dst_v.at[0])
        pltpu.async_copy(h_hbm.at[src_v], rows_v, sem).wait()
        pltpu.sync_copy(rows_v, acc_sh.at[dst_v.at[0]], add=True)
        pltpu.sync_copy(ones_v, dacc_sh.at[dst_v.at[0]], add=True)
        return carry

    lax.fori_loop(0, CHUNKS_PER_W, body, 0)
    plsc.subcore_barrier()
    out_base = c * NP + r0
    pltpu.sync_copy(acc_sh.at[pl.ds(r0, ROWS_PER_SUB)],
                    agg_out.at[pl.ds(out_base, ROWS_PER_SUB)])
    pltpu.sync_copy(dacc_sh.at[pl.ds(r0, ROWS_PER_SUB)],
                    deg_out.at[pl.ds(out_base, ROWS_PER_SUB)])


@functools.cache
def _sc_agg():
    mesh = plsc.VectorSubcoreMesh(core_axis_name="c", subcore_axis_name="s",
                                  num_cores=NCORES, num_subcores=NSUB)
    return pl.kernel(
        _sc_agg_body,
        out_type=(
            jax.ShapeDtypeStruct((NCORES * NP, D), jnp.float32),
            jax.ShapeDtypeStruct((NCORES * NP, DEG_W), jnp.float32),
        ),
        mesh=mesh,
        compiler_params=pltpu.CompilerParams(use_tc_tiling_on_sc=False),
        scratch_types=[
            pltpu.VMEM((CHUNK,), jnp.int32),
            pltpu.VMEM((1, CHUNK), jnp.int32),
            pltpu.VMEM((CHUNK, D), jnp.float32),
            pltpu.VMEM((CHUNK, DEG_W), jnp.float32),
            pltpu.VMEM_SHARED((NP, D), jnp.float32),
            pltpu.VMEM_SHARED((NP, DEG_W), jnp.float32),
            pltpu.SemaphoreType.DMA,
        ],
    )


def kernel(x, edge_index, W1, b1, W2, b2):
    src = edge_index[0].astype(jnp.int32)
    dst = edge_index[1].astype(jnp.int32)
    pad = E_PAD - E
    src = jnp.concatenate([src, jnp.zeros((pad,), jnp.int32)])
    dst = jnp.concatenate([dst, jnp.full((pad,), N, jnp.int32)])
    zagg = jnp.zeros((ROWS_PER_SUB, D), jnp.float32)
    zdeg = jnp.zeros((ROWS_PER_SUB, DEG_W), jnp.float32)
    ones = jnp.ones((CHUNK, DEG_W), jnp.float32)

    h1 = _tc_pre(x, W1, b1, project=True)
    agg1, deg1 = _sc_agg()(h1, src, dst, zagg, zdeg, ones)
    y1 = _tc_post(h1, agg1.reshape(NCORES, NP, D),
                  deg1.reshape(NCORES, NP, DEG_W), activation=True)
    h2 = _tc_pre(y1, W2, b2, project=False)
    agg2, deg2 = _sc_agg()(h2, src, dst, zagg, zdeg, ones)
    out = _tc_post(h2, agg2.reshape(NCORES, NP, D),
                   deg2.reshape(NCORES, NP, DEG_W), activation=False)
    return out
